# Initial kernel scaffold; baseline (speedup 1.0000x reference)
import jax
import jax.numpy as jnp
from jax.experimental import pallas as pl


def kernel(x_circRNA, x_miRNA, x_disease, ei_0, ei_1, ei_2, ei_3, ei_4, ei_5, ei_6, ei_7, W1, a1s, a1d, b1, W2, a2s, a2d, b2):
    raise NotImplementedError("write your pallas kernel here")


# speedup vs baseline: 3.8110x; 3.8110x over previous
import functools

import jax
import jax.numpy as jnp
from jax import lax
from jax.experimental import pallas as pl
from jax.experimental.pallas import tpu as pltpu

_SRC = (0, 1, 0, 1, 2, 2, 0, 2)
_DST = (1, 2, 2, 0, 1, 0, 0, 2)
_BUCKETS = ((3, 5, 6), (0, 4), (1, 2, 7))
_N = 10000
_E = 160000
_R = 1000


def _mm_l1_body(xs_ref, xd_ref, w_ref, wa_ref, h0_ref, h1_ref, al_ref):
    xs = xs_ref[...]
    xd = xd_ref[...]
    w = w_ref[0]
    h = jnp.dot(xs, w, preferred_element_type=jnp.float32)
    rows = xs.shape[0]
    ones = jnp.ones((rows, 2), jnp.float32)
    zeros = jnp.zeros((rows, 14), jnp.float32)
    h0_ref[0] = jnp.concatenate([h[:, 0:128], ones, zeros], axis=1)
    h1_ref[0] = jnp.concatenate([h[:, 128:256], ones, zeros], axis=1)
    as4 = jnp.dot(xs, wa_ref[0][:, 0:4], preferred_element_type=jnp.float32)
    ad4 = jnp.dot(xd, wa_ref[0][:, 4:8], preferred_element_type=jnp.float32)
    al_ref[0] = jnp.concatenate([as4, ad4], axis=1)


def _mm_l2_body(xs_ref, xd_ref, w_ref, wa_ref, h0_ref, h1_ref, al_ref):
    xs = xs_ref[...]
    xd = xd_ref[...]
    w = w_ref[0]
    h = jnp.dot(xs, w, preferred_element_type=jnp.float32)
    rows = xs.shape[0]
    ones = jnp.ones((rows, 1), jnp.float32)
    zeros = jnp.zeros((rows, 15), jnp.float32)
    h0_ref[0] = jnp.concatenate([h[:, 0:32], ones, zeros], axis=1)
    h1_ref[0] = jnp.concatenate([h[:, 32:64], ones, zeros], axis=1)
    as4 = jnp.dot(xs, wa_ref[0][:, 0:4], preferred_element_type=jnp.float32)
    ad4 = jnp.dot(xd, wa_ref[0][:, 4:8], preferred_element_type=jnp.float32)
    al_ref[0] = jnp.concatenate([as4, ad4], axis=1)


def _linear_stage(x3, w, wa, hw, layer1):
    nb = _N // _R

    def _lookup(tab, r):
        return sum(tv * (r == ri) for ri, tv in enumerate(tab) if tv)

    body = _mm_l1_body if layer1 else _mm_l2_body
    k_in = x3.shape[1]
    grid = (8, nb)
    return pl.pallas_call(
        body,
        grid=grid,
        in_specs=[
            pl.BlockSpec((_R, k_in), lambda r, i: (_lookup(_SRC, r) * nb + i, 0)),
            pl.BlockSpec((_R, k_in), lambda r, i: (_lookup(_DST, r) * nb + i, 0)),
            pl.BlockSpec((1, k_in, w.shape[2]), lambda r, i: (r, 0, 0)),
            pl.BlockSpec((1, k_in, 8), lambda r, i: (r, 0, 0)),
        ],
        out_specs=[
            pl.BlockSpec((1, _R, hw), lambda r, i: (r, i, 0)),
            pl.BlockSpec((1, _R, hw), lambda r, i: (r, i, 0)),
            pl.BlockSpec((1, _R, 8), lambda r, i: (r, i, 0)),
        ],
        out_shape=[
            jax.ShapeDtypeStruct((8, _N, hw), jnp.float32),
            jax.ShapeDtypeStruct((8, _N, hw), jnp.float32),
            jax.ShapeDtypeStruct((8, _N, 8), jnp.float32),
        ],
    )(x3, x3, w, wa)


def _combine_l1_body(a0_ref, a1_ref, b_ref, y0_ref, y1_ref, y2_ref):
    outs = [y0_ref, y1_ref, y2_ref]
    for t in range(3):
        acc = None
        for r in _BUCKETS[t]:
            a0 = a0_ref[r]
            a1 = a1_ref[r]
            pieces = []
            for k, a in ((0, a0), (1, a1)):
                s0 = a[:, 128:129] + 1e-16
                s1 = a[:, 129:130] + 1e-16
                pieces.append(a[:, 0:64] / s0)
                pieces.append(a[:, 64:128] / s1)
            o = jnp.concatenate(pieces, axis=1) + b_ref[r][None, :]
            acc = o if acc is None else acc + o
        y = acc / float(len(_BUCKETS[t]))
        y = jnp.where(y > 0, y, jnp.exp(jnp.minimum(y, 0.0)) - 1.0)
        outs[t][...] = y


def _combine_l1(acc0, acc1, b1):
    nb = _N // _R
    return pl.pallas_call(
        _combine_l1_body,
        grid=(nb,),
        in_specs=[
            pl.BlockSpec((8, _R, 144), lambda i: (0, i, 0)),
            pl.BlockSpec((8, _R, 144), lambda i: (0, i, 0)),
            pl.BlockSpec((8, 256), lambda i: (0, 0)),
        ],
        out_specs=[pl.BlockSpec((_R, 256), lambda i: (i, 0))] * 3,
        out_shape=[jax.ShapeDtypeStruct((_N, 256), jnp.float32)] * 3,
    )(acc0, acc1, b1)


def _combine_l2_body(a0_ref, a1_ref, b_ref, y0_ref, y1_ref, y2_ref):
    outs = [y0_ref, y1_ref, y2_ref]
    for t in range(3):
        acc = None
        for r in _BUCKETS[t]:
            a0 = a0_ref[r]
            a1 = a1_ref[r]
            o = jnp.concatenate(
                [a0[:, 0:32] / (a0[:, 32:33] + 1e-16),
                 a1[:, 0:32] / (a1[:, 32:33] + 1e-16)], axis=1)
            o = o + b_ref[r][None, :]
            acc = o if acc is None else acc + o
        y = acc / float(len(_BUCKETS[t]))
        nrm = jnp.sqrt(jnp.sum(y * y, axis=1, keepdims=True))
        y = y / jnp.maximum(nrm, 1e-12)
        outs[t][...] = y


def _combine_l2(acc0, acc1, b2):
    nb = _N // _R
    return pl.pallas_call(
        _combine_l2_body,
        grid=(nb,),
        in_specs=[
            pl.BlockSpec((8, _R, 48), lambda i: (0, i, 0)),
            pl.BlockSpec((8, _R, 48), lambda i: (0, i, 0)),
            pl.BlockSpec((8, 64), lambda i: (0, 0)),
        ],
        out_specs=[pl.BlockSpec((_R, 64), lambda i: (i, 0))] * 3,
        out_shape=[jax.ShapeDtypeStruct((_N, 64), jnp.float32)] * 3,
    )(acc0, acc1, b2)


def _edge_phase_jnp(eis, hsp0, hsp1, alsd, hw, nheads):
    hc = 64 if nheads == 4 else 32
    acc0s, acc1s = [], []
    for r in range(8):
        src = eis[r][0]
        dst = eis[r][1]
        al_s = alsd[r, :, 0:nheads]
        al_d = alsd[r, :, 4:4 + nheads]
        e = al_s[src] + al_d[dst]
        e = jnp.where(e >= 0, e, 0.2 * e)
        ex = jnp.exp(e)
        accs = []
        for k, hsp in ((0, hsp0), (1, hsp1)):
            if nheads == 4:
                gain = jnp.concatenate(
                    [jnp.repeat(ex[:, 2 * k:2 * k + 1], 64, 1),
                     jnp.repeat(ex[:, 2 * k + 1:2 * k + 2], 64, 1),
                     ex[:, 2 * k:2 * k + 2],
                     jnp.zeros((_E, 14), jnp.float32)], axis=1)
            else:
                gain = jnp.concatenate(
                    [jnp.repeat(ex[:, 0:1], 33, 1),
                     jnp.zeros((_E, 15), jnp.float32)], axis=1)
            msg = hsp[r][src] * gain
            accs.append(jax.ops.segment_sum(msg, dst, num_segments=_N))
        acc0s.append(accs[0])
        acc1s.append(accs[1])
    return jnp.stack(acc0s), jnp.stack(acc1s)


def kernel(x_circRNA, x_miRNA, x_disease, ei_0, ei_1, ei_2, ei_3, ei_4,
           ei_5, ei_6, ei_7, W1, a1s, a1d, b1, W2, a2s, a2d, b2):
    eis = [ei_0, ei_1, ei_2, ei_3, ei_4, ei_5, ei_6, ei_7]

    def fold(W, a_s, a_d, H, C):
        Wr = W.reshape(8, W.shape[1], H, C)
        was = jnp.einsum('rihc,rhc->rih', Wr, a_s)
        wad = jnp.einsum('rihc,rhc->rih', Wr, a_d)
        pad = jnp.zeros((8, W.shape[1], 4 - H), jnp.float32)
        return jnp.concatenate([was, pad, wad, pad], axis=2)

    wa1 = fold(W1, a1s, a1d, 4, 64)
    wa2 = fold(W2, a2s, a2d, 1, 64)

    x3 = jnp.concatenate([x_circRNA, x_miRNA, x_disease], axis=0)
    hsp0, hsp1, alsd = _linear_stage(x3, W1, wa1, 144, True)
    acc0, acc1 = _edge_phase_jnp(eis, hsp0, hsp1, alsd, 144, 4)
    y0, y1, y2 = _combine_l1(acc0, acc1, b1)

    x3b = jnp.concatenate([y0, y1, y2], axis=0)
    hsp0b, hsp1b, alsdb = _linear_stage(x3b, W2, wa2, 48, False)
    acc0b, acc1b = _edge_phase_jnp(eis, hsp0b, hsp1b, alsdb, 48, 1)
    z0, z1, z2 = _combine_l2(acc0b, acc1b, b2)
    return (z0, z1, z2)


# speedup vs baseline: 16.1704x; 4.2431x over previous
import functools

import jax
import jax.numpy as jnp
from jax import lax
from jax.experimental import pallas as pl
from jax.experimental.pallas import tpu as pltpu
from jax.experimental.pallas import tpu_sc as plsc

_SRC = (0, 1, 0, 1, 2, 2, 0, 2)
_DST = (1, 2, 2, 0, 1, 0, 0, 2)
_BUCKETS = ((3, 5, 6), (0, 4), (1, 2, 7))
_N = 10000
_E = 160000
_R = 1000


def _mm_l1_body(xs_ref, xd_ref, w_ref, wa_ref, h0_ref, h1_ref, al_ref):
    xs = xs_ref[...]
    xd = xd_ref[...]
    w = w_ref[0]
    h = jnp.dot(xs, w, preferred_element_type=jnp.float32)
    rows = xs.shape[0]
    ones = jnp.ones((rows, 2), jnp.float32)
    zeros = jnp.zeros((rows, 10), jnp.float32)
    as4 = jnp.dot(xs, wa_ref[0][:, 0:4], preferred_element_type=jnp.float32)
    ad4 = jnp.dot(xd, wa_ref[0][:, 4:8], preferred_element_type=jnp.float32)
    h0_ref[0] = jnp.concatenate(
        [h[:, 0:128], ones, as4[:, 0:2], ad4[:, 0:2], zeros], axis=1)
    h1_ref[0] = jnp.concatenate(
        [h[:, 128:256], ones, as4[:, 2:4], ad4[:, 2:4], zeros], axis=1)
    al_ref[0] = jnp.concatenate([as4, ad4], axis=1)


def _mm_l2_body(xs_ref, xd_ref, w_ref, wa_ref, h0_ref, h1_ref, al_ref):
    xs = xs_ref[...]
    xd = xd_ref[...]
    w = w_ref[0]
    h = jnp.dot(xs, w, preferred_element_type=jnp.float32)
    rows = xs.shape[0]
    ones = jnp.ones((rows, 1), jnp.float32)
    zeros = jnp.zeros((rows, 13), jnp.float32)
    as4 = jnp.dot(xs, wa_ref[0][:, 0:4], preferred_element_type=jnp.float32)
    ad4 = jnp.dot(xd, wa_ref[0][:, 4:8], preferred_element_type=jnp.float32)
    h0_ref[0] = jnp.concatenate(
        [h[:, 0:32], ones, as4[:, 0:1], ad4[:, 0:1], zeros], axis=1)
    h1_ref[0] = jnp.concatenate(
        [h[:, 32:64], ones, as4[:, 0:1], ad4[:, 0:1], zeros], axis=1)
    al_ref[0] = jnp.concatenate([as4, ad4], axis=1)


def _linear_stage(x3, w, wa, hw, layer1):
    nb = _N // _R

    def _lookup(tab, r):
        return sum(tv * (r == ri) for ri, tv in enumerate(tab) if tv)

    body = _mm_l1_body if layer1 else _mm_l2_body
    k_in = x3.shape[1]
    grid = (8, nb)
    return pl.pallas_call(
        body,
        grid=grid,
        in_specs=[
            pl.BlockSpec((_R, k_in), lambda r, i: (_lookup(_SRC, r) * nb + i, 0)),
            pl.BlockSpec((_R, k_in), lambda r, i: (_lookup(_DST, r) * nb + i, 0)),
            pl.BlockSpec((1, k_in, w.shape[2]), lambda r, i: (r, 0, 0)),
            pl.BlockSpec((1, k_in, 8), lambda r, i: (r, 0, 0)),
        ],
        out_specs=[
            pl.BlockSpec((1, _R, hw), lambda r, i: (r, i, 0)),
            pl.BlockSpec((1, _R, hw), lambda r, i: (r, i, 0)),
            pl.BlockSpec((1, _R, 8), lambda r, i: (r, i, 0)),
        ],
        out_shape=[
            jax.ShapeDtypeStruct((8, _N, hw), jnp.float32),
            jax.ShapeDtypeStruct((8, _N, hw), jnp.float32),
            jax.ShapeDtypeStruct((8, _N, 8), jnp.float32),
        ],
    )(x3, x3, w, wa)


def _combine_l1_body(a0_ref, a1_ref, b_ref, y0_ref, y1_ref, y2_ref):
    outs = [y0_ref, y1_ref, y2_ref]
    for t in range(3):
        acc = None
        for r in _BUCKETS[t]:
            a0 = a0_ref[r]
            a1 = a1_ref[r]
            pieces = []
            for k, a in ((0, a0), (1, a1)):
                s0 = a[:, 128:129] + 1e-16
                s1 = a[:, 129:130] + 1e-16
                pieces.append(a[:, 0:64] / s0)
                pieces.append(a[:, 64:128] / s1)
            o = jnp.concatenate(pieces, axis=1) + b_ref[r][None, :]
            acc = o if acc is None else acc + o
        y = acc / float(len(_BUCKETS[t]))
        y = jnp.where(y > 0, y, jnp.exp(jnp.minimum(y, 0.0)) - 1.0)
        outs[t][...] = y


def _combine_l1(acc0, acc1, b1):
    nb = _N // _R
    return pl.pallas_call(
        _combine_l1_body,
        grid=(nb,),
        in_specs=[
            pl.BlockSpec((8, _R, 144), lambda i: (0, i, 0)),
            pl.BlockSpec((8, _R, 144), lambda i: (0, i, 0)),
            pl.BlockSpec((8, 256), lambda i: (0, 0)),
        ],
        out_specs=[pl.BlockSpec((_R, 256), lambda i: (i, 0))] * 3,
        out_shape=[jax.ShapeDtypeStruct((_N, 256), jnp.float32)] * 3,
    )(acc0, acc1, b1)


def _combine_l2_body(a0_ref, a1_ref, b_ref, y0_ref, y1_ref, y2_ref):
    outs = [y0_ref, y1_ref, y2_ref]
    for t in range(3):
        acc = None
        for r in _BUCKETS[t]:
            a0 = a0_ref[r]
            a1 = a1_ref[r]
            o = jnp.concatenate(
                [a0[:, 0:32] / (a0[:, 32:33] + 1e-16),
                 a1[:, 0:32] / (a1[:, 32:33] + 1e-16)], axis=1)
            o = o + b_ref[r][None, :]
            acc = o if acc is None else acc + o
        y = acc / float(len(_BUCKETS[t]))
        nrm = jnp.sqrt(jnp.sum(y * y, axis=1, keepdims=True))
        y = y / jnp.maximum(nrm, 1e-12)
        outs[t][...] = y


def _combine_l2(acc0, acc1, b2):
    nb = _N // _R
    return pl.pallas_call(
        _combine_l2_body,
        grid=(nb,),
        in_specs=[
            pl.BlockSpec((8, _R, 48), lambda i: (0, i, 0)),
            pl.BlockSpec((8, _R, 48), lambda i: (0, i, 0)),
            pl.BlockSpec((8, 64), lambda i: (0, 0)),
        ],
        out_specs=[pl.BlockSpec((_R, 64), lambda i: (i, 0))] * 3,
        out_shape=[jax.ShapeDtypeStruct((_N, 64), jnp.float32)] * 3,
    )(acc0, acc1, b2)


_B = 80
_NBAT = 128
_EPT = 10000
_NP = 10112
_RPT = 632
_CHK = 64


def _edge_body(hw, nh, scol, gsrc_hbm, dst_hbm, hs_hbm, out_hbm,
               src_v, dst_v, gidx_v, gdst_v, dbuf, msg, acc_sp, sem):
    k = lax.axis_index("c")
    t = lax.axis_index("s")
    nvr = hw // 16
    zero16 = jnp.zeros((16,), jnp.float32)
    iot = lax.iota(jnp.int32, 16)
    m0 = jnp.where(iot == 0, 1.0, 0.0)
    m1 = jnp.where(iot == 1, 1.0, 0.0)
    c0 = iot * 0
    c1 = c0 + 1
    c2 = c0 + 2
    c3 = c0 + 3

    def relation(r, _):
        plsc.subcore_barrier()
        def zrow(i, _):
            for vr in range(nvr):
                msg[i, pl.ds(vr * 16, 16)] = zero16
            return 0

        lax.fori_loop(0, _B, zrow, 0)
        for i in range(7):
            pltpu.sync_copy(msg, acc_sp.at[pl.ds(t * _RPT + i * _B, _B)])
        pltpu.sync_copy(msg.at[pl.ds(0, 72)],
                        acc_sp.at[pl.ds(t * _RPT + 560, 72)])

        plsc.subcore_barrier()

        def chunk(c, _):
            rowbase = (r * 16 + t) * _NBAT + c * _CHK
            pltpu.sync_copy(gsrc_hbm.at[pl.ds(rowbase, _CHK)], src_v)
            pltpu.sync_copy(dst_hbm.at[pl.ds(rowbase, _CHK)], dst_v)

            def batch(j, _):
                for g in range(_B // 16):
                    sl = pl.ds(g * 16, 16)
                    gidx_v[sl] = src_v[j, sl] + (k * 8 + r) * _N
                    gdst_v[sl] = dst_v[j, sl] + (k * 8 + r) * _N
                pltpu.async_copy(hs_hbm.at[gdst_v], dbuf, sem).wait()
                pltpu.async_copy(hs_hbm.at[gidx_v], msg, sem).wait()
                for e in range(_B):
                    eb = c0 + e
                    a0 = plsc.load_gather(msg, [eb, c0 + scol])
                    b0 = plsc.load_gather(dbuf, [eb, c0 + scol + nh])
                    e0 = a0 + b0
                    e0 = jnp.maximum(e0, 0.2 * e0)
                    ex0 = jnp.exp(e0)
                    if nh == 2:
                        a1 = plsc.load_gather(msg, [eb, c0 + scol + 1])
                        b1 = plsc.load_gather(dbuf, [eb, c0 + scol + 3])
                        e1 = a1 + b1
                        e1 = jnp.maximum(e1, 0.2 * e1)
                        ex1 = jnp.exp(e1)
                        mix = ex0 * m0 + ex1 * m1
                        for v in range(nvr):
                            sl = pl.ds(v * 16, 16)
                            if v < 4:
                                msg[e, sl] = msg[e, sl] * ex0
                            elif v < 8:
                                msg[e, sl] = msg[e, sl] * ex1
                            else:
                                msg[e, sl] = msg[e, sl] * mix
                    else:
                        for v in range(nvr):
                            sl = pl.ds(v * 16, 16)
                            msg[e, sl] = msg[e, sl] * ex0

                pltpu.sync_copy(msg, acc_sp.at[dst_v.at[j]], add=True)
                return 0

            lax.fori_loop(0, _CHK, batch, 0)
            return 0

        lax.fori_loop(0, _NBAT // _CHK, chunk, 0)
        plsc.subcore_barrier()
        pltpu.sync_copy(acc_sp.at[pl.ds(t * _RPT, _RPT)],
                        out_hbm.at[pl.ds((k * 8 + r) * _NP + t * _RPT, _RPT)])
        return 0

    lax.fori_loop(0, 8, relation, 0)


def _edge_phase_sc(eis, hsp0, hsp1, hw, nheads):
    nh = 2 if nheads == 4 else 1
    scol = 130 if nheads == 4 else 33
    hs_flat = jnp.concatenate(
        [hsp0.reshape(8 * _N, hw), hsp1.reshape(8 * _N, hw),
         jnp.zeros((16, hw), jnp.float32)], axis=0)
    npad = _NBAT * _B - _EPT
    pad_src = (jnp.arange(npad, dtype=jnp.int32) * 131) % _N
    pad_dst = _N + jnp.arange(npad, dtype=jnp.int32) % 16
    gsrc_list, dst_list = [], []
    for r in range(8):
        s = eis[r][0].reshape(16, _EPT)
        d = eis[r][1].reshape(16, _EPT)
        s = jnp.concatenate([s, jnp.tile(pad_src[None], (16, 1))], axis=1)
        d = jnp.concatenate([d, jnp.tile(pad_dst[None], (16, 1))], axis=1)
        gsrc_list.append(s)
        dst_list.append(d)
    gsrc = jnp.stack(gsrc_list).reshape(8 * 16 * _NBAT, _B)
    dstv = jnp.stack(dst_list).reshape(8 * 16 * _NBAT, _B)

    mesh = plsc.VectorSubcoreMesh(core_axis_name="c", subcore_axis_name="s")
    fn = pl.kernel(
        functools.partial(_edge_body, hw, nh, scol),
        out_type=jax.ShapeDtypeStruct((16 * _NP, hw), jnp.float32),
        mesh=mesh,
        compiler_params=pltpu.CompilerParams(use_tc_tiling_on_sc=False, needs_layout_passes=False),
        scratch_types=[
            pltpu.VMEM((_CHK, _B), jnp.int32),
            pltpu.VMEM((_CHK, _B), jnp.int32),
            pltpu.VMEM((_B,), jnp.int32),
            pltpu.VMEM((_B,), jnp.int32),
            pltpu.VMEM((_B, hw), jnp.float32),
            pltpu.VMEM((_B, hw), jnp.float32),
            pltpu.VMEM_SHARED((_NP, hw), jnp.float32),
            pltpu.SemaphoreType.DMA,
        ],
    )
    out = fn(gsrc, dstv, hs_flat)
    out = out.reshape(2, 8, _NP, hw)
    return out[0], out[1]


def _edge_phase_jnp(eis, hsp0, hsp1, alsd, hw, nheads, exp2=False):
    hc = 64 if nheads == 4 else 32
    acc0s, acc1s = [], []
    for r in range(8):
        src = eis[r][0]
        dst = eis[r][1]
        al_s = alsd[r, :, 0:nheads]
        al_d = alsd[r, :, 4:4 + nheads]
        e = al_s[src] + al_d[dst]
        e = jnp.where(e >= 0, e, 0.2 * e)
        ex = e if exp2 == "noexp" else jnp.exp(e)
        accs = []
        for k, hsp in ((0, hsp0), (1, hsp1)):
            if nheads == 4:
                gain = jnp.concatenate(
                    [jnp.repeat(ex[:, 2 * k:2 * k + 1], 64, 1),
                     jnp.repeat(ex[:, 2 * k + 1:2 * k + 2], 64, 1),
                     ex[:, 2 * k:2 * k + 2],
                     jnp.zeros((_E, 14), jnp.float32)], axis=1)
            else:
                gain = jnp.concatenate(
                    [jnp.repeat(ex[:, 0:1], 33, 1),
                     jnp.zeros((_E, 15), jnp.float32)], axis=1)
            msg = hsp[r][src] * gain
            accs.append(jax.ops.segment_sum(msg, dst, num_segments=_N))
        acc0s.append(accs[0])
        acc1s.append(accs[1])
    return jnp.stack(acc0s), jnp.stack(acc1s)


def kernel(x_circRNA, x_miRNA, x_disease, ei_0, ei_1, ei_2, ei_3, ei_4,
           ei_5, ei_6, ei_7, W1, a1s, a1d, b1, W2, a2s, a2d, b2):
    eis = [ei_0, ei_1, ei_2, ei_3, ei_4, ei_5, ei_6, ei_7]

    def fold(W, a_s, a_d, H, C):
        Wr = W.reshape(8, W.shape[1], H, C)
        was = jnp.einsum('rihc,rhc->rih', Wr, a_s)
        wad = jnp.einsum('rihc,rhc->rih', Wr, a_d)
        pad = jnp.zeros((8, W.shape[1], 4 - H), jnp.float32)
        return jnp.concatenate([was, pad, wad, pad], axis=2)

    wa1 = fold(W1, a1s, a1d, 4, 64)
    wa2 = fold(W2, a2s, a2d, 1, 64)

    x3 = jnp.concatenate([x_circRNA, x_miRNA, x_disease], axis=0)
    hsp0, hsp1, alsd = _linear_stage(x3, W1, wa1, 144, True)
    acc0, acc1 = _edge_phase_sc(eis, hsp0, hsp1, 144, 4)
    y0, y1, y2 = _combine_l1(acc0, acc1, b1)

    x3b = jnp.concatenate([y0, y1, y2], axis=0)
    hsp0b, hsp1b, alsdb = _linear_stage(x3b, W2, wa2, 48, False)
    acc0b, acc1b = _edge_phase_sc(eis, hsp0b, hsp1b, 48, 1)
    z0, z1, z2 = _combine_l2(acc0b, acc1b, b2)
    return (z0, z1, z2)


# speedup vs baseline: 16.1883x; 1.0011x over previous
import functools

import jax
import jax.numpy as jnp
from jax import lax
from jax.experimental import pallas as pl
from jax.experimental.pallas import tpu as pltpu
from jax.experimental.pallas import tpu_sc as plsc

_SRC = (0, 1, 0, 1, 2, 2, 0, 2)
_DST = (1, 2, 2, 0, 1, 0, 0, 2)
_BUCKETS = ((3, 5, 6), (0, 4), (1, 2, 7))
_N = 10000
_E = 160000
_R = 1000


def _mm_l1_body(xs_ref, xd_ref, w_ref, wa_ref, h0_ref, h1_ref, al_ref):
    xs = xs_ref[...]
    xd = xd_ref[...]
    w = w_ref[0]
    h = jnp.dot(xs, w, preferred_element_type=jnp.float32)
    rows = xs.shape[0]
    ones = jnp.ones((rows, 2), jnp.float32)
    zeros = jnp.zeros((rows, 10), jnp.float32)
    as4 = jnp.dot(xs, wa_ref[0][:, 0:4], preferred_element_type=jnp.float32)
    ad4 = jnp.dot(xd, wa_ref[0][:, 4:8], preferred_element_type=jnp.float32)
    h0_ref[0] = jnp.concatenate(
        [h[:, 0:128], ones, as4[:, 0:2], ad4[:, 0:2], zeros], axis=1)
    h1_ref[0] = jnp.concatenate(
        [h[:, 128:256], ones, as4[:, 2:4], ad4[:, 2:4], zeros], axis=1)
    al_ref[0] = jnp.concatenate([as4, ad4], axis=1)


def _mm_l2_body(xs_ref, xd_ref, w_ref, wa_ref, h0_ref, h1_ref, al_ref):
    xs = xs_ref[...]
    xd = xd_ref[...]
    w = w_ref[0]
    h = jnp.dot(xs, w, preferred_element_type=jnp.float32)
    rows = xs.shape[0]
    ones = jnp.ones((rows, 1), jnp.float32)
    zeros = jnp.zeros((rows, 13), jnp.float32)
    as4 = jnp.dot(xs, wa_ref[0][:, 0:4], preferred_element_type=jnp.float32)
    ad4 = jnp.dot(xd, wa_ref[0][:, 4:8], preferred_element_type=jnp.float32)
    h0_ref[0] = jnp.concatenate(
        [h[:, 0:32], ones, as4[:, 0:1], ad4[:, 0:1], zeros], axis=1)
    h1_ref[0] = jnp.concatenate(
        [h[:, 32:64], ones, as4[:, 0:1], ad4[:, 0:1], zeros], axis=1)
    al_ref[0] = jnp.concatenate([as4, ad4], axis=1)


def _linear_stage(x3, w, wa, hw, layer1):
    nb = _N // _R

    def _lookup(tab, r):
        return sum(tv * (r == ri) for ri, tv in enumerate(tab) if tv)

    body = _mm_l1_body if layer1 else _mm_l2_body
    k_in = x3.shape[1]
    grid = (8, nb)
    return pl.pallas_call(
        body,
        grid=grid,
        in_specs=[
            pl.BlockSpec((_R, k_in), lambda r, i: (_lookup(_SRC, r) * nb + i, 0)),
            pl.BlockSpec((_R, k_in), lambda r, i: (_lookup(_DST, r) * nb + i, 0)),
            pl.BlockSpec((1, k_in, w.shape[2]), lambda r, i: (r, 0, 0)),
            pl.BlockSpec((1, k_in, 8), lambda r, i: (r, 0, 0)),
        ],
        out_specs=[
            pl.BlockSpec((1, _R, hw), lambda r, i: (r, i, 0)),
            pl.BlockSpec((1, _R, hw), lambda r, i: (r, i, 0)),
            pl.BlockSpec((1, _R, 8), lambda r, i: (r, i, 0)),
        ],
        out_shape=[
            jax.ShapeDtypeStruct((8, _N, hw), jnp.float32),
            jax.ShapeDtypeStruct((8, _N, hw), jnp.float32),
            jax.ShapeDtypeStruct((8, _N, 8), jnp.float32),
        ],
    )(x3, x3, w, wa)


def _combine_l1_body(a0_ref, a1_ref, b_ref, y0_ref, y1_ref, y2_ref):
    outs = [y0_ref, y1_ref, y2_ref]
    for t in range(3):
        acc = None
        for r in _BUCKETS[t]:
            a0 = a0_ref[r]
            a1 = a1_ref[r]
            pieces = []
            for k, a in ((0, a0), (1, a1)):
                s0 = a[:, 128:129] + 1e-16
                s1 = a[:, 129:130] + 1e-16
                pieces.append(a[:, 0:64] / s0)
                pieces.append(a[:, 64:128] / s1)
            o = jnp.concatenate(pieces, axis=1) + b_ref[r][None, :]
            acc = o if acc is None else acc + o
        y = acc / float(len(_BUCKETS[t]))
        y = jnp.where(y > 0, y, jnp.exp(jnp.minimum(y, 0.0)) - 1.0)
        outs[t][...] = y


def _combine_l1(acc0, acc1, b1):
    nb = _N // _R
    return pl.pallas_call(
        _combine_l1_body,
        grid=(nb,),
        in_specs=[
            pl.BlockSpec((8, _R, 144), lambda i: (0, i, 0)),
            pl.BlockSpec((8, _R, 144), lambda i: (0, i, 0)),
            pl.BlockSpec((8, 256), lambda i: (0, 0)),
        ],
        out_specs=[pl.BlockSpec((_R, 256), lambda i: (i, 0))] * 3,
        out_shape=[jax.ShapeDtypeStruct((_N, 256), jnp.float32)] * 3,
    )(acc0, acc1, b1)


def _combine_l2_body(a0_ref, a1_ref, b_ref, y0_ref, y1_ref, y2_ref):
    outs = [y0_ref, y1_ref, y2_ref]
    for t in range(3):
        acc = None
        for r in _BUCKETS[t]:
            a0 = a0_ref[r]
            a1 = a1_ref[r]
            o = jnp.concatenate(
                [a0[:, 0:32] / (a0[:, 32:33] + 1e-16),
                 a1[:, 0:32] / (a1[:, 32:33] + 1e-16)], axis=1)
            o = o + b_ref[r][None, :]
            acc = o if acc is None else acc + o
        y = acc / float(len(_BUCKETS[t]))
        nrm = jnp.sqrt(jnp.sum(y * y, axis=1, keepdims=True))
        y = y / jnp.maximum(nrm, 1e-12)
        outs[t][...] = y


def _combine_l2(acc0, acc1, b2):
    nb = _N // _R
    return pl.pallas_call(
        _combine_l2_body,
        grid=(nb,),
        in_specs=[
            pl.BlockSpec((8, _R, 48), lambda i: (0, i, 0)),
            pl.BlockSpec((8, _R, 48), lambda i: (0, i, 0)),
            pl.BlockSpec((8, 64), lambda i: (0, 0)),
        ],
        out_specs=[pl.BlockSpec((_R, 64), lambda i: (i, 0))] * 3,
        out_shape=[jax.ShapeDtypeStruct((_N, 64), jnp.float32)] * 3,
    )(acc0, acc1, b2)


_B = 80
_NBAT = 128
_EPT = 10000
_NP = 10112
_RPT = 632
_CHK = 64


def _edge_body(hw, nh, scol, gsrc_hbm, dst_hbm, hs_hbm, out_hbm,
               src_v, dst_v, gidx_v, gdst_v, dbuf, msg, acc_sp, sem):
    k = lax.axis_index("c")
    t = lax.axis_index("s")
    nvr = hw // 16
    zero16 = jnp.zeros((16,), jnp.float32)
    iot = lax.iota(jnp.int32, 16)
    m0 = jnp.where(iot == 0, 1.0, 0.0)
    m1 = jnp.where(iot == 1, 1.0, 0.0)
    c0 = iot * 0
    c1 = c0 + 1
    c2 = c0 + 2
    c3 = c0 + 3

    def relation(r, _):
        plsc.subcore_barrier()
        def zrow(i, _):
            for vr in range(nvr):
                msg[i, pl.ds(vr * 16, 16)] = zero16
            return 0

        lax.fori_loop(0, _B, zrow, 0)
        for i in range(7):
            pltpu.sync_copy(msg, acc_sp.at[pl.ds(t * _RPT + i * _B, _B)])
        pltpu.sync_copy(msg.at[pl.ds(0, 72)],
                        acc_sp.at[pl.ds(t * _RPT + 560, 72)])

        plsc.subcore_barrier()

        def chunk(c, _):
            rowbase = (r * 16 + t) * _NBAT + c * _CHK
            pltpu.sync_copy(gsrc_hbm.at[pl.ds(rowbase, _CHK)], src_v)
            pltpu.sync_copy(dst_hbm.at[pl.ds(rowbase, _CHK)], dst_v)

            def batch(j, _):
                for g in range(_B // 16):
                    sl = pl.ds(g * 16, 16)
                    gidx_v[sl] = src_v[j, sl] + (k * 8 + r) * _N
                    gdst_v[sl] = dst_v[j, sl] + (k * 8 + r) * _N
                pltpu.async_copy(hs_hbm.at[gdst_v], dbuf, sem).wait()
                pltpu.async_copy(hs_hbm.at[gidx_v], msg, sem).wait()
                for e in range(_B):
                    eb = c0 + e
                    a0 = plsc.load_gather(msg, [eb, c0 + scol])
                    b0 = plsc.load_gather(dbuf, [eb, c0 + scol + nh])
                    e0 = a0 + b0
                    e0 = jnp.maximum(e0, 0.2 * e0)
                    ex0 = jnp.exp(e0)
                    if nh == 2:
                        a1 = plsc.load_gather(msg, [eb, c0 + scol + 1])
                        b1 = plsc.load_gather(dbuf, [eb, c0 + scol + 3])
                        e1 = a1 + b1
                        e1 = jnp.maximum(e1, 0.2 * e1)
                        ex1 = jnp.exp(e1)
                        mix = ex0 * m0 + ex1 * m1
                        for v in range(nvr):
                            sl = pl.ds(v * 16, 16)
                            if v < 4:
                                msg[e, sl] = msg[e, sl] * ex0
                            elif v < 8:
                                msg[e, sl] = msg[e, sl] * ex1
                            else:
                                msg[e, sl] = msg[e, sl] * mix
                    else:
                        for v in range(nvr):
                            sl = pl.ds(v * 16, 16)
                            msg[e, sl] = msg[e, sl] * ex0

                pltpu.sync_copy(msg, acc_sp.at[dst_v.at[j]], add=True)
                return 0

            lax.fori_loop(0, _CHK, batch, 0)
            return 0

        lax.fori_loop(0, _NBAT // _CHK, chunk, 0)
        plsc.subcore_barrier()
        pltpu.sync_copy(acc_sp.at[pl.ds(t * _RPT, _RPT)],
                        out_hbm.at[pl.ds((k * 8 + r) * _NP + t * _RPT, _RPT)])
        return 0

    lax.fori_loop(0, 8, relation, 0)


def _edge_phase_sc(eis, hsp0, hsp1, hw, nheads):
    nh = 2 if nheads == 4 else 1
    scol = 130 if nheads == 4 else 33
    hs_flat = jnp.concatenate(
        [hsp0.reshape(8 * _N, hw), hsp1.reshape(8 * _N, hw),
         jnp.zeros((16, hw), jnp.float32)], axis=0)
    npad = _NBAT * _B - _EPT
    pad_src = (jnp.arange(npad, dtype=jnp.int32) * 131) % _N
    pad_dst = _N + jnp.arange(npad, dtype=jnp.int32) % 16
    gsrc_list, dst_list = [], []
    for r in range(8):
        s = eis[r][0].reshape(16, _EPT)
        d = eis[r][1].reshape(16, _EPT)
        s = jnp.concatenate([s, jnp.tile(pad_src[None], (16, 1))], axis=1)
        d = jnp.concatenate([d, jnp.tile(pad_dst[None], (16, 1))], axis=1)
        gsrc_list.append(s)
        dst_list.append(d)
    gsrc = jnp.stack(gsrc_list).reshape(8 * 16 * _NBAT, _B)
    dstv = jnp.stack(dst_list).reshape(8 * 16 * _NBAT, _B)

    mesh = plsc.VectorSubcoreMesh(core_axis_name="c", subcore_axis_name="s")
    fn = pl.kernel(
        functools.partial(_edge_body, hw, nh, scol),
        out_type=jax.ShapeDtypeStruct((16 * _NP, hw), jnp.float32),
        mesh=mesh,
        compiler_params=pltpu.CompilerParams(use_tc_tiling_on_sc=False, needs_layout_passes=False),
        scratch_types=[
            pltpu.VMEM((_CHK, _B), jnp.int32),
            pltpu.VMEM((_CHK, _B), jnp.int32),
            pltpu.VMEM((_B,), jnp.int32),
            pltpu.VMEM((_B,), jnp.int32),
            pltpu.VMEM((_B, hw), jnp.float32),
            pltpu.VMEM((_B, hw), jnp.float32),
            pltpu.VMEM_SHARED((_NP, hw), jnp.float32),
            pltpu.SemaphoreType.DMA,
        ],
    )
    out = fn(gsrc, dstv, hs_flat)
    out = out.reshape(2, 8, _NP, hw)
    return out[0], out[1]


def kernel(x_circRNA, x_miRNA, x_disease, ei_0, ei_1, ei_2, ei_3, ei_4,
           ei_5, ei_6, ei_7, W1, a1s, a1d, b1, W2, a2s, a2d, b2):
    eis = [ei_0, ei_1, ei_2, ei_3, ei_4, ei_5, ei_6, ei_7]

    def fold(W, a_s, a_d, H, C):
        Wr = W.reshape(8, W.shape[1], H, C)
        was = jnp.einsum('rihc,rhc->rih', Wr, a_s)
        wad = jnp.einsum('rihc,rhc->rih', Wr, a_d)
        pad = jnp.zeros((8, W.shape[1], 4 - H), jnp.float32)
        return jnp.concatenate([was, pad, wad, pad], axis=2)

    wa1 = fold(W1, a1s, a1d, 4, 64)
    wa2 = fold(W2, a2s, a2d, 1, 64)

    x3 = jnp.concatenate([x_circRNA, x_miRNA, x_disease], axis=0)
    hsp0, hsp1, alsd = _linear_stage(x3, W1, wa1, 144, True)
    acc0, acc1 = _edge_phase_sc(eis, hsp0, hsp1, 144, 4)
    y0, y1, y2 = _combine_l1(acc0, acc1, b1)

    x3b = jnp.concatenate([y0, y1, y2], axis=0)
    hsp0b, hsp1b, alsdb = _linear_stage(x3b, W2, wa2, 48, False)
    acc0b, acc1b = _edge_phase_sc(eis, hsp0b, hsp1b, 48, 1)
    z0, z1, z2 = _combine_l2(acc0b, acc1b, b2)
    return (z0, z1, z2)
